# baseline (speedup 1.0000x reference)
import functools

import jax
import jax.numpy as jnp
import numpy as np
from jax import lax
from jax.experimental import pallas as pl
from jax.experimental.pallas import tpu as pltpu
from jax.experimental.pallas import tpu_sc as plsc

B, F, L, V, D = 1024, 26, 20, 100000, 16
NC, NS = 2, 16
NW = NC * NS
ROWS = F * B * L
RPW = ROWS // NW
IDXW = 128
IRPW = RPW // IDXW
BAGS_PW = RPW // L
CB = 64
CROWS = CB * L
NCHUNK = BAGS_PW // CB
GPC = CROWS // IDXW

_mesh = plsc.VectorSubcoreMesh(
    core_axis_name="c", subcore_axis_name="s", num_cores=NC, num_subcores=NS
)


@functools.partial(
    pl.kernel,
    out_type=(
        jax.ShapeDtypeStruct((ROWS, D), jnp.float32),
        jax.ShapeDtypeStruct((B * F, D), jnp.float32),
    ),
    mesh=_mesh,
    compiler_params=pltpu.CompilerParams(use_tc_tiling_on_sc=False),
    scratch_types=(
        pltpu.VMEM((IRPW, IDXW), jnp.int32),
        pltpu.VMEM((NCHUNK, CB), jnp.int32),
        pltpu.VMEM((CROWS, D), jnp.float32),
        pltpu.VMEM((CROWS, D), jnp.float32),
        pltpu.VMEM((CB, D), jnp.float32),
        pltpu.SemaphoreType.DMA,
        pltpu.SemaphoreType.DMA,
    ),
)
def _sparse_arch_sc(idx_hbm, orow_hbm, ebc_t, ec_t, ec_out, ebc_out,
                    idx_v, orow_v, ebc_buf, ec_buf, pooled_v, sem_e, sem_c):
    wid = lax.axis_index("s") * NC + lax.axis_index("c")
    pltpu.sync_copy(idx_hbm.at[wid], idx_v)
    pltpu.sync_copy(orow_hbm.at[wid], orow_v)

    def chunk_body(c, carry):
        ebc_dmas = []
        ec_dmas = []
        for j in range(GPC):
            r = c * GPC + j
            ebc_dmas.append(pltpu.async_copy(
                ebc_t.at[idx_v.at[r]], ebc_buf.at[pl.ds(j * IDXW, IDXW)], sem_e))
            ec_dmas.append(pltpu.async_copy(
                ec_t.at[idx_v.at[r]], ec_buf.at[pl.ds(j * IDXW, IDXW)], sem_c))
        for d in ec_dmas:
            d.wait()
        pltpu.sync_copy(ec_buf, ec_out.at[pl.ds(wid * RPW + c * CROWS, CROWS)])
        for d in ebc_dmas:
            d.wait()

        def bag_body(jb, carry2):
            base = jb * L
            acc = ebc_buf[base]
            for l in range(1, L):
                acc = acc + ebc_buf[base + l]
            pooled_v[jb] = acc
            return carry2

        lax.fori_loop(0, CB, bag_body, 0, unroll=False)
        pltpu.sync_copy(pooled_v, ebc_out.at[orow_v.at[c]])
        return carry

    lax.fori_loop(0, NCHUNK, chunk_body, 0, unroll=False)


def _pooled_out_rows() -> np.ndarray:
    g = np.arange(F * B, dtype=np.int32)
    rows = (g % B) * F + (g // B)
    return rows.reshape(NW, NCHUNK, CB)


def kernel(indices, ebc_tables, ec_tables):
    offs = (jnp.arange(F, dtype=jnp.int32) * V)[:, None, None]
    idx_fm = (jnp.transpose(indices, (1, 0, 2)).astype(jnp.int32) + offs)
    idx_fm = idx_fm.reshape(NW, IRPW, IDXW)
    orow = jnp.asarray(_pooled_out_rows())
    ec_values, pooled = _sparse_arch_sc(
        idx_fm, orow,
        ebc_tables.reshape(F * V, D), ec_tables.reshape(F * V, D))
    return pooled.reshape(B, F * D), ec_values
